# Initial kernel scaffold; baseline (speedup 1.0000x reference)
import jax
import jax.numpy as jnp
from jax.experimental import pallas as pl


def kernel(users, items, inter, adj_indices, adj_values, user_emb, item_emb, W_gc, b_gc, W_bi, b_bi, inter_W, inter_b, pred_W, pred_b):
    raise NotImplementedError("write your pallas kernel here")


# speedup vs baseline: 3.0674x; 3.0674x over previous
import functools

import jax
import jax.numpy as jnp
from jax import lax
from jax.experimental import pallas as pl
from jax.experimental.pallas import tpu as pltpu
from jax.experimental.pallas import tpu_sc as plsc

N_USER = 25000
N_ITEM = 25000
NN = N_USER + N_ITEM
E = 64
NNZ = 800000
BATCH = 4096
CI = 16
NL = 3

HALF = NN // 2
HALF_PAD = 25600
ROWS_PER_TILE = HALF_PAD // 16

BLK = 256
SUB = 128
NSUB = BLK // SUB
NBLK = 196
EPT = NBLK * BLK
NNZ_PAD = 16 * EPT

_mesh = plsc.VectorSubcoreMesh(core_axis_name="c", subcore_axis_name="s")
_params = pltpu.CompilerParams(use_tc_tiling_on_sc=False)


@functools.partial(
    pl.kernel,
    out_type=jax.ShapeDtypeStruct((NN, E), jnp.float32),
    mesh=_mesh,
    compiler_params=_params,
    scratch_types=[
        pltpu.VMEM((BLK,), jnp.float32),
        pltpu.VMEM((NSUB, SUB), jnp.int32),
        pltpu.VMEM((NSUB, SUB), jnp.int32),
        pltpu.VMEM((BLK, E), jnp.float32),
        pltpu.VMEM_SHARED((HALF_PAD, E), jnp.float32),
        pltpu.SemaphoreType.DMA,
    ],
)
def _spmv(ego, colr, s0r, s1r, v0r, v1r, zrows, out,
          val_v, cidx, sidx, rows3, acc, sem):
    c = lax.axis_index("c")
    s = lax.axis_index("s")
    lo = c * HALF

    pltpu.sync_copy(zrows, acc.at[pl.ds(s * ROWS_PER_TILE, ROWS_PER_TILE)])
    plsc.subcore_barrier()

    def _blk_body(blk, carry):
        base = s * EPT + blk * BLK
        for sb in range(NSUB):
            pltpu.sync_copy(colr.at[pl.ds(base + sb * SUB, SUB)], cidx.at[sb])

        @pl.when(c == 0)
        def _():
            pltpu.sync_copy(v0r.at[pl.ds(base, BLK)], val_v)
            for sb in range(NSUB):
                pltpu.sync_copy(s0r.at[pl.ds(base + sb * SUB, SUB)],
                                sidx.at[sb])

        @pl.when(c == 1)
        def _():
            pltpu.sync_copy(v1r.at[pl.ds(base, BLK)], val_v)
            for sb in range(NSUB):
                pltpu.sync_copy(s1r.at[pl.ds(base + sb * SUB, SUB)],
                                sidx.at[sb])

        handles = [pltpu.async_copy(ego.at[cidx.at[sb]],
                                    rows3.at[pl.ds(sb * SUB, SUB)], sem)
                   for sb in range(NSUB)]
        for h in handles:
            h.wait()

        def _scale(i, carry2):
            vv = val_v[pl.ds(i * 16, 16)]
            for j in range(16):
                r = i * 16 + j
                v = vv[j]
                for q in range(E // 16):
                    t = rows3[r, pl.ds(q * 16, 16)]
                    rows3[r, pl.ds(q * 16, 16)] = t * v
            return carry2

        lax.fori_loop(0, BLK // 16, _scale, 0)
        for sb in range(NSUB):
            pltpu.sync_copy(rows3.at[pl.ds(sb * SUB, SUB)],
                            acc.at[sidx.at[sb]], add=True)
        return carry

    lax.fori_loop(0, NBLK, _blk_body, 0)

    plsc.subcore_barrier()
    for i in range(ROWS_PER_TILE // 200):
        start = s * ROWS_PER_TILE + i * 200

        @pl.when(start < HALF)
        def _():
            pltpu.sync_copy(acc.at[pl.ds(start, 200)],
                            out.at[pl.ds(lo + start, 200)])


@functools.partial(
    pl.kernel,
    out_type=jax.ShapeDtypeStruct((4, 2 * BATCH, E), jnp.float32),
    mesh=_mesh,
    compiler_params=_params,
    scratch_types=[
        pltpu.VMEM((2, SUB), jnp.int32),
        pltpu.VMEM((SUB, E), jnp.float32),
        pltpu.SemaphoreType.DMA,
    ],
)
def _gather4(e0, e1, e2, e3, gixr, out, gix, gbuf, sem):
    c = lax.axis_index("c")
    s = lax.axis_index("s")
    w = s * 2 + c
    base = w * (2 * BATCH // 32)
    for j in range(2):
        pltpu.sync_copy(gixr.at[pl.ds(base + j * SUB, SUB)], gix.at[j])
    for k, tab in enumerate((e0, e1, e2, e3)):
        for j in range(2):
            pltpu.async_copy(tab.at[gix.at[j]], gbuf, sem).wait()
            pltpu.sync_copy(gbuf, out.at[k, pl.ds(base + j * SUB, SUB)])


def _layer_body(ego_ref, side_ref, wg_ref, bg_ref, wb_ref, bb_ref,
                ego_out_ref, nrm_out_ref):
    ego = ego_ref[...]
    side = side_ref[...]
    x = (jnp.dot(side, wg_ref[...], preferred_element_type=jnp.float32)
         + bg_ref[...]
         + jnp.dot(ego * side, wb_ref[...], preferred_element_type=jnp.float32)
         + bb_ref[...])
    x = jnp.where(x >= 0, x, 0.2 * x)
    ego_out_ref[...] = x
    nrm = jnp.sqrt(jnp.sum(x * x, axis=1, keepdims=True))
    nrm_out_ref[...] = x / jnp.maximum(nrm, 1e-12)


_RB = 400


def _layer(ego, side, wg, bg, wb, bb):
    spec_rows = pl.BlockSpec((_RB, E), lambda i: (i, 0))
    spec_w = pl.BlockSpec((E, E), lambda i: (0, 0))
    spec_b = pl.BlockSpec((1, E), lambda i: (0, 0))
    return pl.pallas_call(
        _layer_body,
        grid=(NN // _RB,),
        in_specs=[spec_rows, spec_rows, spec_w, spec_b, spec_w, spec_b],
        out_specs=[spec_rows, spec_rows],
        out_shape=[jax.ShapeDtypeStruct((NN, E), jnp.float32)] * 2,
    )(ego, side, wg, bg, wb, bb)


def _head_body(g_ref, inter_ref, iwt_ref, ib_ref, wp_ref, wi_ref, pb_ref,
               o_ref):
    g = g_ref[...]
    wp = wp_ref[...]
    acc = jnp.zeros((BATCH, 1), jnp.float32)
    for k in range(4):
        pair = g[k, :BATCH, :] * g[k, BATCH:, :]
        acc = acc + jnp.dot(pair, wp[k].reshape(E, 1),
                            preferred_element_type=jnp.float32)
    it = jnp.dot(inter_ref[...], iwt_ref[...],
                 preferred_element_type=jnp.float32) + ib_ref[...]
    acc = acc + jnp.dot(it, wi_ref[...], preferred_element_type=jnp.float32)
    acc = acc + pb_ref[...]
    o_ref[...] = jax.nn.sigmoid(acc)


def _head(g, inter, iwt, ib, wp, wi, pb):
    whole = lambda *shape: pl.BlockSpec(shape, lambda: tuple(0 for _ in shape))
    return pl.pallas_call(
        _head_body,
        in_specs=[whole(4, 2 * BATCH, E), whole(BATCH, CI), whole(CI, CI),
                  whole(1, CI), whole(4, E), whole(CI, 1), whole(1, 1)],
        out_specs=whole(BATCH, 1),
        out_shape=jax.ShapeDtypeStruct((BATCH, 1), jnp.float32),
    )(g, inter, iwt, ib, wp, wi, pb)


def kernel(users, items, inter, adj_indices, adj_values, user_emb, item_emb,
           W_gc, b_gc, W_bi, b_bi, inter_W, inter_b, pred_W, pred_b):
    row = adj_indices[0].astype(jnp.int32)
    col = adj_indices[1].astype(jnp.int32)
    vals = adj_values.astype(jnp.float32)

    pad = NNZ_PAD - NNZ
    i_all = jnp.arange(NNZ_PAD, dtype=jnp.int32)
    dmy = HALF + (i_all & 15)
    rowp = jnp.concatenate([row, jnp.full((pad,), NN, jnp.int32)])
    colp = jnp.concatenate([col, i_all[:pad] % NN])
    valp = jnp.concatenate([vals, jnp.zeros((pad,), jnp.float32)])
    m0 = rowp < HALF
    m1 = (rowp >= HALF) & (rowp < NN)
    s0 = jnp.where(m0, rowp, dmy)
    s1 = jnp.where(m1, rowp - HALF, dmy)
    v0 = jnp.where(m0, valp, 0.0)
    v1 = jnp.where(m1, valp, 0.0)
    zrows = jnp.zeros((ROWS_PER_TILE, E), jnp.float32)

    ego = jnp.concatenate([user_emb, item_emb], axis=0)
    tabs = [ego]
    for k in range(NL):
        side = _spmv(ego, colp, s0, s1, v0, v1, zrows)
        ego, nrm = _layer(ego, side, W_gc[k], b_gc[k], W_bi[k], b_bi[k])
        tabs.append(nrm)
    idx_all = jnp.concatenate([users.astype(jnp.int32),
                               items.astype(jnp.int32) + N_USER])
    g = _gather4(tabs[0], tabs[1], tabs[2], tabs[3], idx_all)
    pw = pred_W.reshape(-1)
    wp = pw[:4 * E].reshape(4, E)
    wi = pw[4 * E:].reshape(CI, 1)
    out = _head(g, inter, inter_W.T, inter_b.reshape(1, CI), wp, wi,
                pred_b.reshape(1, 1))
    return out.reshape(-1)
